# baseline (speedup 1.0000x reference)
import functools

import jax
import jax.numpy as jnp
from jax import lax
from jax.experimental import pallas as pl
from jax.experimental.pallas import tpu as pltpu
from jax.experimental.pallas import tpu_sc as plsc

BATCH = 16384
EMBED_DIM = 50
DPAD = 64
NC, NS = 2, 16
NW = NC * NS
NSPLIT = 2
HB = BATCH // NSPLIT
BPW = HB // NW
WAVE = 64
NWAVE = BPW // WAVE
NBUF = 8
BLK = 2048
H1, H2 = 128, 64

_mesh = plsc.VectorSubcoreMesh(core_axis_name="c", subcore_axis_name="s")


@functools.partial(
    pl.kernel,
    out_type=[
        jax.ShapeDtypeStruct((HB * DPAD,), jnp.float32),
        jax.ShapeDtypeStruct((HB * DPAD,), jnp.float32),
    ],
    mesh=_mesh,
    scratch_types=[
        pltpu.VMEM((BPW,), jnp.int32),
        pltpu.VMEM((BPW,), jnp.int32),
        pltpu.VMEM((NBUF, EMBED_DIM, 128), jnp.float32),
        pltpu.VMEM((2 * WAVE * DPAD,), jnp.float32),
        pltpu.SemaphoreType.DMA,
        pltpu.SemaphoreType.DMA,
        pltpu.SemaphoreType.DMA,
        pltpu.SemaphoreType.DMA,
        pltpu.SemaphoreType.DMA,
        pltpu.SemaphoreType.DMA,
        pltpu.SemaphoreType.DMA,
        pltpu.SemaphoreType.DMA,
    ],
    compiler_params=pltpu.CompilerParams(needs_layout_passes=False),
)
def _sc_gather(user_hbm, item_hbm, ut_t_hbm, it_t_hbm, ue_out, ie_out,
               uidx_v, iidx_v, stripes_v, rowbuf_v,
               s0, s1, s2, s3, s4, s5, s6, s7):
    wid = lax.axis_index("s") * NC + lax.axis_index("c")
    base = wid * BPW
    sems = [s0, s1, s2, s3, s4, s5, s6, s7]
    pltpu.sync_copy(user_hbm.at[pl.ds(base, BPW)], uidx_v)
    pltpu.sync_copy(item_hbm.at[pl.ds(base, BPW)], iidx_v)

    row_ids = jnp.minimum(lax.iota(jnp.int32, 16), EMBED_DIM - 1)

    def table_sweep(idx_v, tab_hbm, out_hbm):
        def scal(start, k):
            vec = idx_v[pl.ds(start + (k // 16) * 16, 16)]
            v = vec[k % 16]
            return v // 128, v % 128

        def fire(slot, start, k, dep):
            tc, _ = scal(start, k)
            tc = tc + dep
            pltpu.async_copy(
                tab_hbm.at[:, pl.ds(pl.multiple_of(tc * 128, 128), 128)],
                stripes_v.at[slot], sems[slot])

        def wait_extract(slot, start, k, half, j):
            pltpu.make_async_copy(
                tab_hbm.at[:, pl.ds(0, 128)],
                stripes_v.at[slot], sems[slot]).wait()
            _, lane = scal(start, k)
            lanes = jnp.full((16,), lane, jnp.int32)
            rbase = half * (WAVE * DPAD) + j * DPAD
            for t in range(DPAD // 16):
                vals = plsc.load_gather(
                    stripes_v.at[slot],
                    [jnp.minimum(row_ids + 16 * t, EMBED_DIM - 1), lanes])
                rowbuf_v[pl.ds(rbase + 16 * t, 16)] = vals

        def flush(gprev):
            half = lax.rem(gprev, 2)
            pltpu.sync_copy(
                rowbuf_v.at[pl.ds(half * (WAVE * DPAD), WAVE * DPAD)],
                out_hbm.at[pl.ds((base + gprev * WAVE) * DPAD, WAVE * DPAD)])

        def rb_dep(half, j):
            w = rowbuf_v[pl.ds(half * (WAVE * DPAD) + j * DPAD + DPAD - 16, 16)]
            return jnp.sum(w.astype(jnp.int32) * 0)

        def wave_body(g, carry):
            half_g = lax.rem(g, 2)
            for k in range(WAVE):
                slot = k % NBUF
                if k < NBUF:
                    @pl.when(g >= 1)
                    def _():
                        wait_extract(slot, (g - 1) * WAVE, WAVE - NBUF + k,
                                     1 - half_g, WAVE - NBUF + k)
                    dep = rb_dep(1 - half_g, WAVE - NBUF + k)
                else:
                    wait_extract(slot, g * WAVE, k - NBUF, half_g, k - NBUF)
                    dep = rb_dep(half_g, k - NBUF)
                if k == NBUF - 1:
                    @pl.when(g >= 1)
                    def _():
                        flush(g - 1)
                fire(slot, g * WAVE, k, dep)
            return carry

        lax.fori_loop(0, NWAVE, wave_body, 0)
        for e in range(NBUF):
            wait_extract(e, (NWAVE - 1) * WAVE, WAVE - NBUF + e,
                         (NWAVE - 1) % 2, WAVE - NBUF + e)
        flush(NWAVE - 1)

    table_sweep(uidx_v, ut_t_hbm, ue_out)
    table_sweep(iidx_v, it_t_hbm, ie_out)


def _mlp_body(ue_ref, ie_ref, w1u_ref, w1i_ref, b1_ref, w2_ref, b2_ref,
              w3_ref, b3_ref, out_ref):
    h = (jnp.dot(ue_ref[...], w1u_ref[...], preferred_element_type=jnp.float32)
         + jnp.dot(ie_ref[...], w1i_ref[...], preferred_element_type=jnp.float32)
         + b1_ref[...])
    h = jnp.maximum(h, 0.0)
    h = jnp.maximum(
        jnp.dot(h, w2_ref[...], preferred_element_type=jnp.float32) + b2_ref[...],
        0.0)
    r = jnp.sum(h * w3_ref[...], axis=1) + b3_ref[0, 0]
    out_ref[...] = r


_mlp = pl.pallas_call(
    _mlp_body,
    grid=(HB // BLK,),
    in_specs=[
        pl.BlockSpec((BLK, DPAD), lambda i: (i, 0)),
        pl.BlockSpec((BLK, DPAD), lambda i: (i, 0)),
        pl.BlockSpec((DPAD, H1), lambda i: (0, 0)),
        pl.BlockSpec((DPAD, H1), lambda i: (0, 0)),
        pl.BlockSpec((1, H1), lambda i: (0, 0)),
        pl.BlockSpec((H1, H2), lambda i: (0, 0)),
        pl.BlockSpec((1, H2), lambda i: (0, 0)),
        pl.BlockSpec((1, H2), lambda i: (0, 0)),
        pl.BlockSpec((1, 1), lambda i: (0, 0)),
    ],
    out_specs=pl.BlockSpec((BLK,), lambda i: (i,)),
    out_shape=jax.ShapeDtypeStruct((HB,), jnp.float32),
)


def kernel(user, item, user_table, item_table, W1, b1, W2, b2, W3, b3):
    ut_t, it_t = user_table.T, item_table.T
    pad = ((0, DPAD - EMBED_DIM), (0, 0))
    w1u, w1i = jnp.pad(W1[:EMBED_DIM], pad), jnp.pad(W1[EMBED_DIM:], pad)
    b1r, b2r = b1.reshape(1, H1), b2.reshape(1, H2)
    w3r, b3r = W3.reshape(1, H2), b3.reshape(1, 1)
    outs = []
    for h in range(NSPLIT):
        sl = slice(h * HB, (h + 1) * HB)
        u1d, i1d = _sc_gather(user[sl], item[sl], ut_t, it_t)
        outs.append(_mlp(u1d.reshape(HB, DPAD), i1d.reshape(HB, DPAD),
                         w1u, w1i, b1r, W2, b2r, w3r, b3r))
    return jnp.concatenate(outs)


# speedup vs baseline: 1.0488x; 1.0488x over previous
import functools

import jax
import jax.numpy as jnp
from jax import lax
from jax.experimental import pallas as pl
from jax.experimental.pallas import tpu as pltpu
from jax.experimental.pallas import tpu_sc as plsc

BATCH = 16384
EMBED_DIM = 50
DPAD = 64
NC, NS = 2, 16
NW = NC * NS
BPW = BATCH // NW
WAVE = 64
NWAVE = BPW // WAVE
NBUF = 16
BLK = 2048
H1, H2 = 128, 64

_mesh = plsc.VectorSubcoreMesh(core_axis_name="c", subcore_axis_name="s")


@functools.partial(
    pl.kernel,
    out_type=[
        jax.ShapeDtypeStruct((BATCH * DPAD,), jnp.float32),
        jax.ShapeDtypeStruct((BATCH * DPAD,), jnp.float32),
    ],
    mesh=_mesh,
    scratch_types=[
        pltpu.VMEM((BPW,), jnp.int32),
        pltpu.VMEM((BPW,), jnp.int32),
        pltpu.VMEM((NBUF, EMBED_DIM, 128), jnp.float32),
        pltpu.VMEM((2 * WAVE * DPAD,), jnp.float32),
        pltpu.SemaphoreType.DMA,
        pltpu.SemaphoreType.DMA,
        pltpu.SemaphoreType.DMA,
        pltpu.SemaphoreType.DMA,
        pltpu.SemaphoreType.DMA,
        pltpu.SemaphoreType.DMA,
        pltpu.SemaphoreType.DMA,
        pltpu.SemaphoreType.DMA,
        pltpu.SemaphoreType.DMA,
        pltpu.SemaphoreType.DMA,
        pltpu.SemaphoreType.DMA,
        pltpu.SemaphoreType.DMA,
        pltpu.SemaphoreType.DMA,
        pltpu.SemaphoreType.DMA,
        pltpu.SemaphoreType.DMA,
        pltpu.SemaphoreType.DMA,
    ],
    compiler_params=pltpu.CompilerParams(needs_layout_passes=False),
)
def _sc_gather(user_hbm, item_hbm, ut_t_hbm, it_t_hbm, ue_out, ie_out,
               uidx_v, iidx_v, stripes_v, rowbuf_v,
               s0, s1, s2, s3, s4, s5, s6, s7,
               s8, s9, s10, s11, s12, s13, s14, s15):
    wid = lax.axis_index("s") * NC + lax.axis_index("c")
    base = wid * BPW
    sems = [s0, s1, s2, s3, s4, s5, s6, s7,
            s8, s9, s10, s11, s12, s13, s14, s15]
    pltpu.sync_copy(user_hbm.at[pl.ds(base, BPW)], uidx_v)
    pltpu.sync_copy(item_hbm.at[pl.ds(base, BPW)], iidx_v)

    row_ids = jnp.minimum(lax.iota(jnp.int32, 16), EMBED_DIM - 1)

    def table_sweep(idx_v, tab_hbm, out_hbm):
        def scal(start, k):
            vec = idx_v[pl.ds(start + (k // 16) * 16, 16)]
            v = vec[k % 16]
            return v // 128, v % 128

        def fire(slot, start, k, dep):
            tc, _ = scal(start, k)
            tc = tc + dep
            pltpu.async_copy(
                tab_hbm.at[:, pl.ds(pl.multiple_of(tc * 128, 128), 128)],
                stripes_v.at[slot], sems[slot])

        def wait_extract(slot, start, k, half, j):
            pltpu.make_async_copy(
                tab_hbm.at[:, pl.ds(0, 128)],
                stripes_v.at[slot], sems[slot]).wait()
            _, lane = scal(start, k)
            lanes = jnp.full((16,), lane, jnp.int32)
            rbase = half * (WAVE * DPAD) + j * DPAD
            for t in range(DPAD // 16):
                vals = plsc.load_gather(
                    stripes_v.at[slot],
                    [jnp.minimum(row_ids + 16 * t, EMBED_DIM - 1), lanes])
                rowbuf_v[pl.ds(rbase + 16 * t, 16)] = vals

        def flush(gprev):
            half = lax.rem(gprev, 2)
            pltpu.sync_copy(
                rowbuf_v.at[pl.ds(half * (WAVE * DPAD), WAVE * DPAD)],
                out_hbm.at[pl.ds((base + gprev * WAVE) * DPAD, WAVE * DPAD)])

        def rb_dep(half, j):
            w = rowbuf_v[pl.ds(half * (WAVE * DPAD) + j * DPAD + DPAD - 16, 16)]
            return jnp.sum(w.astype(jnp.int32) * 0)

        def wave_body(g, carry):
            half_g = lax.rem(g, 2)
            for k in range(WAVE):
                slot = k % NBUF
                if k < NBUF:
                    @pl.when(g >= 1)
                    def _():
                        wait_extract(slot, (g - 1) * WAVE, WAVE - NBUF + k,
                                     1 - half_g, WAVE - NBUF + k)
                    dep = rb_dep(1 - half_g, WAVE - NBUF + k)
                else:
                    wait_extract(slot, g * WAVE, k - NBUF, half_g, k - NBUF)
                    dep = rb_dep(half_g, k - NBUF)
                if k == NBUF - 1:
                    @pl.when(g >= 1)
                    def _():
                        flush(g - 1)
                fire(slot, g * WAVE, k, dep)
            return carry

        lax.fori_loop(0, NWAVE, wave_body, 0)
        for e in range(NBUF):
            wait_extract(e, (NWAVE - 1) * WAVE, WAVE - NBUF + e,
                         (NWAVE - 1) % 2, WAVE - NBUF + e)
        flush(NWAVE - 1)

    table_sweep(uidx_v, ut_t_hbm, ue_out)
    table_sweep(iidx_v, it_t_hbm, ie_out)


def _mlp_body(ue_ref, ie_ref, w1u_ref, w1i_ref, b1_ref, w2_ref, b2_ref,
              w3_ref, b3_ref, out_ref):
    h = (jnp.dot(ue_ref[...], w1u_ref[...], preferred_element_type=jnp.float32)
         + jnp.dot(ie_ref[...], w1i_ref[...], preferred_element_type=jnp.float32)
         + b1_ref[...])
    h = jnp.maximum(h, 0.0)
    h = jnp.maximum(
        jnp.dot(h, w2_ref[...], preferred_element_type=jnp.float32) + b2_ref[...],
        0.0)
    r = jnp.sum(h * w3_ref[...], axis=1) + b3_ref[0, 0]
    out_ref[...] = r


_mlp = pl.pallas_call(
    _mlp_body,
    grid=(BATCH // BLK,),
    in_specs=[
        pl.BlockSpec((BLK, DPAD), lambda i: (i, 0)),
        pl.BlockSpec((BLK, DPAD), lambda i: (i, 0)),
        pl.BlockSpec((DPAD, H1), lambda i: (0, 0)),
        pl.BlockSpec((DPAD, H1), lambda i: (0, 0)),
        pl.BlockSpec((1, H1), lambda i: (0, 0)),
        pl.BlockSpec((H1, H2), lambda i: (0, 0)),
        pl.BlockSpec((1, H2), lambda i: (0, 0)),
        pl.BlockSpec((1, H2), lambda i: (0, 0)),
        pl.BlockSpec((1, 1), lambda i: (0, 0)),
    ],
    out_specs=pl.BlockSpec((BLK,), lambda i: (i,)),
    out_shape=jax.ShapeDtypeStruct((BATCH,), jnp.float32),
)


def kernel(user, item, user_table, item_table, W1, b1, W2, b2, W3, b3):
    u1d, i1d = _sc_gather(user, item, user_table.T, item_table.T)
    ue_w = u1d.reshape(BATCH, DPAD)
    ie_w = i1d.reshape(BATCH, DPAD)
    pad = ((0, DPAD - EMBED_DIM), (0, 0))
    return _mlp(ue_w, ie_w,
                jnp.pad(W1[:EMBED_DIM], pad), jnp.pad(W1[EMBED_DIM:], pad),
                b1.reshape(1, H1),
                W2, b2.reshape(1, H2),
                W3.reshape(1, H2), b3.reshape(1, 1))
